# baseline (speedup 1.0000x reference)
import functools

import jax
import jax.numpy as jnp
from jax import lax
from jax.experimental import pallas as pl
from jax.experimental.pallas import tpu as pltpu
from jax.experimental.pallas import tpu_sc as plsc

N = 10000
E = 320000
F_IN = 128
DIM = 256
G = 64
BN_EPS = 1e-4

NC = 2
NS = 16
HALF = N // NC
C = 80
NCHUNK = 250
PAIRS = NCHUNK // 2
EPT = NCHUNK * C
EPAD = EPT * NS
RPT = 312
LAST = HALF - (NS - 1) * RPT
TRASH0 = HALF
ACC_ROWS = TRASH0 + NS


def _seg_body(h_hbm, src_hbm, dst_hbm, zeros_hbm, out_hbm,
              sA, dA, dlA, rA, sB, dB, dlB, rB, acc_sh,
              isemA, isemB, gsemA, gsemB, ssemA, ssemB):
    cid = lax.axis_index("c")
    sid = lax.axis_index("s")
    base_node = cid * HALF
    @pl.when(sid < NS - 1)
    def _():
        pltpu.sync_copy(zeros_hbm.at[pl.ds(0, RPT)],
                        acc_sh.at[pl.ds(sid * RPT, RPT)])

    @pl.when(sid == NS - 1)
    def _():
        pltpu.sync_copy(zeros_hbm.at[pl.ds(0, LAST)],
                        acc_sh.at[pl.ds((NS - 1) * RPT, LAST)])
    plsc.subcore_barrier()
    trash = TRASH0 + sid
    e0 = sid * EPT

    def idx_start(j, sv, dv, isem):
        eb = e0 + j * C
        pltpu.async_copy(src_hbm.at[pl.ds(eb, C)], sv, isem)
        pltpu.async_copy(dst_hbm.at[pl.ds(eb, C)], dv, isem)

    def idx_wait(sv, dv, isem):
        pltpu.make_async_copy(src_hbm.at[pl.ds(e0, C)], sv, isem).wait()
        pltpu.make_async_copy(dst_hbm.at[pl.ds(e0, C)], dv, isem).wait()

    def remap(dv, dlv, n):
        for i in range(n // 16):
            d = dv[pl.ds(i * 16, 16)]
            loc = d - base_node
            inb = (loc >= 0) & (loc < HALF)
            dlv[pl.ds(i * 16, 16)] = jnp.where(inb, loc, trash)

    def gath_start(sv, rv, gsem):
        pltpu.async_copy(h_hbm.at[sv], rv, gsem)

    def gath_wait(sv, rv, gsem):
        pltpu.make_async_copy(h_hbm.at[sv], rv, gsem).wait()

    def scat_start(rv, dlv, ssem):
        pltpu.async_copy(rv, acc_sh.at[dlv], ssem, add=True, priority=1)

    def scat_wait(rv, dlv, ssem):
        pltpu.make_async_copy(rv, acc_sh.at[dlv], ssem).wait()

    idx_start(0, sA, dA, isemA)
    idx_start(1, sB, dB, isemB)
    idx_wait(sA, dA, isemA)
    remap(dA, dlA, C)
    gath_start(sA, rA, gsemA)

    @pl.loop(0, PAIRS)
    def _pair(k):
        gath_wait(sA, rA, gsemA)
        scat_start(rA, dlA, ssemA)

        @pl.when(k + 1 < PAIRS)
        def _():
            idx_start(2 * k + 2, sA, dA, isemA)

        idx_wait(sB, dB, isemB)

        @pl.when(k > 0)
        def _():
            scat_wait(rB, dlB, ssemB)
        remap(dB, dlB, C)
        gath_start(sB, rB, gsemB)

        gath_wait(sB, rB, gsemB)
        scat_start(rB, dlB, ssemB)

        @pl.when(k + 1 < PAIRS)
        def _():
            idx_start(2 * k + 3, sB, dB, isemB)
            idx_wait(sA, dA, isemA)
            scat_wait(rA, dlA, ssemA)
            remap(dA, dlA, C)
            gath_start(sA, rA, gsemA)

    scat_wait(rA, dlA, ssemA)
    scat_wait(rB, dlB, ssemB)

    plsc.subcore_barrier()

    @pl.when(sid < NS - 1)
    def _():
        pltpu.sync_copy(acc_sh.at[pl.ds(sid * RPT, RPT)],
                        out_hbm.at[pl.ds(base_node + sid * RPT, RPT)])

    @pl.when(sid == NS - 1)
    def _():
        pltpu.sync_copy(acc_sh.at[pl.ds((NS - 1) * RPT, LAST)],
                        out_hbm.at[pl.ds(base_node + (NS - 1) * RPT, LAST)])


_seg_kernel = functools.partial(
    pl.kernel,
    _seg_body,
    out_type=jax.ShapeDtypeStruct((N, DIM), jnp.float32),
    mesh=plsc.VectorSubcoreMesh(core_axis_name="c", subcore_axis_name="s",
                                num_cores=NC, num_subcores=NS),
    compiler_params=pltpu.CompilerParams(use_tc_tiling_on_sc=False),
    scratch_types=[
        pltpu.VMEM((C,), jnp.int32),
        pltpu.VMEM((C,), jnp.int32),
        pltpu.VMEM((C,), jnp.int32),
        pltpu.VMEM((C, DIM), jnp.float32),
        pltpu.VMEM((C,), jnp.int32),
        pltpu.VMEM((C,), jnp.int32),
        pltpu.VMEM((C,), jnp.int32),
        pltpu.VMEM((C, DIM), jnp.float32),
        pltpu.VMEM_SHARED((ACC_ROWS, DIM), jnp.float32),
        pltpu.SemaphoreType.DMA,
        pltpu.SemaphoreType.DMA,
        pltpu.SemaphoreType.DMA,
        pltpu.SemaphoreType.DMA,
        pltpu.SemaphoreType.DMA,
        pltpu.SemaphoreType.DMA,
    ],
)


def _segment_sum(h, src, dst, zeros):
    return _seg_kernel()(h, src, dst, zeros)


def _pre_body(x_ref, w_ref, b_ref, o_ref):
    o_ref[...] = (jnp.dot(x_ref[...], w_ref[...],
                          preferred_element_type=jnp.float32) + b_ref[...])


def _pre_linear(x, W, b):
    return pl.pallas_call(
        _pre_body,
        out_shape=jax.ShapeDtypeStruct((N, DIM), jnp.float32),
    )(x, W, b.reshape(1, DIM))


def _layer_body(h_ref, a_ref, w1_ref, b1_ref, w2_ref, b2_ref, g_ref, be_ref,
                bf_ref, z_ref, p_ref):
    z0 = h_ref[...] + a_ref[...]
    z1 = jnp.dot(z0, w1_ref[...], preferred_element_type=jnp.float32) + b1_ref[...]
    z1 = jnp.where(z1 > 0, z1, 0.01 * z1)
    z2 = jnp.dot(z1, w2_ref[...], preferred_element_type=jnp.float32) + b2_ref[...]
    mu = jnp.mean(z2, axis=0, keepdims=True)
    zc = z2 - mu
    var = jnp.mean(zc * zc, axis=0, keepdims=True)
    zn = zc * lax.rsqrt(var + BN_EPS) * g_ref[...] + be_ref[...]
    z_ref[...] = zn
    oh = (lax.broadcasted_iota(jnp.int32, (G, N), 0) == bf_ref[...]
          ).astype(jnp.float32)
    p_ref[...] = jnp.dot(oh, zn, preferred_element_type=jnp.float32)


def _layer(h, agg, W1, b1, W2, b2, gam, bet, batchf):
    return pl.pallas_call(
        _layer_body,
        out_shape=(jax.ShapeDtypeStruct((N, DIM), jnp.float32),
                   jax.ShapeDtypeStruct((G, DIM), jnp.float32)),
    )(h, agg, W1, b1.reshape(1, DIM), W2, b2.reshape(1, DIM),
      gam.reshape(1, DIM), bet.reshape(1, DIM), batchf)


def kernel(x, edge_index, batch, W_pre, b_pre,
           W1_0, b1_0, W2_0, b2_0, gam_0, bet_0,
           W1_1, b1_1, W2_1, b2_1, gam_1, bet_1,
           W1_2, b1_2, W2_2, b2_2, gam_2, bet_2):
    pad = EPAD - E
    src = jnp.concatenate([edge_index[0], (jnp.arange(pad, dtype=jnp.int32) * 37) % N])
    dst = jnp.concatenate([edge_index[1], jnp.full((pad,), N, jnp.int32)])
    batchf = batch.reshape(1, N)
    zeros = jnp.zeros((LAST, DIM), jnp.float32)
    layers = [
        (W1_0, b1_0, W2_0, b2_0, gam_0, bet_0),
        (W1_1, b1_1, W2_1, b2_1, gam_1, bet_1),
        (W1_2, b1_2, W2_2, b2_2, gam_2, bet_2),
    ]
    h = _pre_linear(x, W_pre, b_pre)
    xs, pools = [], []
    for (W1, b1, W2, b2, gam, bet) in layers:
        agg = _segment_sum(h, src, dst, zeros)
        h, p = _layer(h, agg, W1, b1, W2, b2, gam, bet, batchf)
        xs.append(h)
        pools.append(p)
    return (jnp.concatenate(pools, axis=1), jnp.concatenate(xs, axis=1))


# speedup vs baseline: 1.0039x; 1.0039x over previous
import functools

import jax
import jax.numpy as jnp
from jax import lax
from jax.experimental import pallas as pl
from jax.experimental.pallas import tpu as pltpu
from jax.experimental.pallas import tpu_sc as plsc

N = 10000
E = 320000
F_IN = 128
DIM = 256
G = 64
BN_EPS = 1e-4

NC = 2
NS = 16
HALF = N // NC
C = 80
NCHUNK = 250
PAIRS = NCHUNK // 2
EPT = NCHUNK * C
RPT = 312
LAST = HALF - (NS - 1) * RPT
TRASH0 = HALF
ACC_ROWS = TRASH0 + NS


def _seg_body(h_hbm, src_hbm, dst_hbm, zeros_hbm, out_hbm,
              sA, dA, dlA, rA, sB, dB, dlB, rB, acc_sh,
              isemA, isemB, gsemA, gsemB, ssemA, ssemB):
    cid = lax.axis_index("c")
    sid = lax.axis_index("s")
    base_node = cid * HALF
    @pl.when(sid < NS - 1)
    def _():
        pltpu.sync_copy(zeros_hbm.at[pl.ds(0, RPT)],
                        acc_sh.at[pl.ds(sid * RPT, RPT)])

    @pl.when(sid == NS - 1)
    def _():
        pltpu.sync_copy(zeros_hbm.at[pl.ds(0, LAST)],
                        acc_sh.at[pl.ds((NS - 1) * RPT, LAST)])
    plsc.subcore_barrier()
    trash = TRASH0 + sid
    e0 = sid * EPT

    def idx_start(j, sv, dv, isem):
        eb = e0 + j * C
        pltpu.async_copy(src_hbm.at[pl.ds(eb, C)], sv, isem)
        pltpu.async_copy(dst_hbm.at[pl.ds(eb, C)], dv, isem)

    def idx_wait(sv, dv, isem):
        pltpu.make_async_copy(src_hbm.at[pl.ds(e0, C)], sv, isem).wait()
        pltpu.make_async_copy(dst_hbm.at[pl.ds(e0, C)], dv, isem).wait()

    def remap(dv, dlv, n):
        for i in range(n // 16):
            d = dv[pl.ds(i * 16, 16)]
            loc = d - base_node
            inb = (loc >= 0) & (loc < HALF)
            dlv[pl.ds(i * 16, 16)] = jnp.where(inb, loc, trash)

    def gath_start(sv, rv, gsem):
        pltpu.async_copy(h_hbm.at[sv], rv, gsem)

    def gath_wait(sv, rv, gsem):
        pltpu.make_async_copy(h_hbm.at[sv], rv, gsem).wait()

    def scat_start(rv, dlv, ssem):
        pltpu.async_copy(rv, acc_sh.at[dlv], ssem, add=True)

    def scat_wait(rv, dlv, ssem):
        pltpu.make_async_copy(rv, acc_sh.at[dlv], ssem).wait()

    idx_start(0, sA, dA, isemA)
    idx_start(1, sB, dB, isemB)
    idx_wait(sA, dA, isemA)
    remap(dA, dlA, C)
    gath_start(sA, rA, gsemA)

    @pl.loop(0, PAIRS)
    def _pair(k):
        gath_wait(sA, rA, gsemA)
        scat_start(rA, dlA, ssemA)

        @pl.when(k + 1 < PAIRS)
        def _():
            idx_start(2 * k + 2, sA, dA, isemA)

        idx_wait(sB, dB, isemB)

        @pl.when(k > 0)
        def _():
            scat_wait(rB, dlB, ssemB)
        remap(dB, dlB, C)
        gath_start(sB, rB, gsemB)

        gath_wait(sB, rB, gsemB)
        scat_start(rB, dlB, ssemB)

        @pl.when(k + 1 < PAIRS)
        def _():
            idx_start(2 * k + 3, sB, dB, isemB)
            idx_wait(sA, dA, isemA)
            scat_wait(rA, dlA, ssemA)
            remap(dA, dlA, C)
            gath_start(sA, rA, gsemA)

    scat_wait(rA, dlA, ssemA)
    scat_wait(rB, dlB, ssemB)

    plsc.subcore_barrier()

    @pl.when(sid < NS - 1)
    def _():
        pltpu.sync_copy(acc_sh.at[pl.ds(sid * RPT, RPT)],
                        out_hbm.at[pl.ds(base_node + sid * RPT, RPT)])

    @pl.when(sid == NS - 1)
    def _():
        pltpu.sync_copy(acc_sh.at[pl.ds((NS - 1) * RPT, LAST)],
                        out_hbm.at[pl.ds(base_node + (NS - 1) * RPT, LAST)])


_seg_kernel = functools.partial(
    pl.kernel,
    _seg_body,
    out_type=jax.ShapeDtypeStruct((N, DIM), jnp.float32),
    mesh=plsc.VectorSubcoreMesh(core_axis_name="c", subcore_axis_name="s",
                                num_cores=NC, num_subcores=NS),
    compiler_params=pltpu.CompilerParams(use_tc_tiling_on_sc=False),
    scratch_types=[
        pltpu.VMEM((C,), jnp.int32),
        pltpu.VMEM((C,), jnp.int32),
        pltpu.VMEM((C,), jnp.int32),
        pltpu.VMEM((C, DIM), jnp.float32),
        pltpu.VMEM((C,), jnp.int32),
        pltpu.VMEM((C,), jnp.int32),
        pltpu.VMEM((C,), jnp.int32),
        pltpu.VMEM((C, DIM), jnp.float32),
        pltpu.VMEM_SHARED((ACC_ROWS, DIM), jnp.float32),
        pltpu.SemaphoreType.DMA,
        pltpu.SemaphoreType.DMA,
        pltpu.SemaphoreType.DMA,
        pltpu.SemaphoreType.DMA,
        pltpu.SemaphoreType.DMA,
        pltpu.SemaphoreType.DMA,
    ],
)


def _segment_sum(h, src, dst, zeros):
    return _seg_kernel()(h, src, dst, zeros)


def _pre_body(x_ref, w_ref, b_ref, o_ref):
    o_ref[...] = (jnp.dot(x_ref[...], w_ref[...],
                          preferred_element_type=jnp.float32) + b_ref[...])


def _pre_linear(x, W, b):
    return pl.pallas_call(
        _pre_body,
        out_shape=jax.ShapeDtypeStruct((N, DIM), jnp.float32),
    )(x, W, b.reshape(1, DIM))


def _layer_body(h_ref, a_ref, w1_ref, b1_ref, w2_ref, b2_ref, g_ref, be_ref,
                bf_ref, z_ref, p_ref):
    z0 = h_ref[...] + a_ref[...]
    z1 = jnp.dot(z0, w1_ref[...], preferred_element_type=jnp.float32) + b1_ref[...]
    z1 = jnp.where(z1 > 0, z1, 0.01 * z1)
    z2 = jnp.dot(z1, w2_ref[...], preferred_element_type=jnp.float32) + b2_ref[...]
    mu = jnp.mean(z2, axis=0, keepdims=True)
    zc = z2 - mu
    var = jnp.mean(zc * zc, axis=0, keepdims=True)
    zn = zc * lax.rsqrt(var + BN_EPS) * g_ref[...] + be_ref[...]
    z_ref[...] = zn
    oh = (lax.broadcasted_iota(jnp.int32, (G, N), 0) == bf_ref[...]
          ).astype(jnp.float32)
    p_ref[...] = jnp.dot(oh, zn, preferred_element_type=jnp.float32)


def _layer(h, agg, W1, b1, W2, b2, gam, bet, batchf):
    return pl.pallas_call(
        _layer_body,
        out_shape=(jax.ShapeDtypeStruct((N, DIM), jnp.float32),
                   jax.ShapeDtypeStruct((G, DIM), jnp.float32)),
    )(h, agg, W1, b1.reshape(1, DIM), W2, b2.reshape(1, DIM),
      gam.reshape(1, DIM), bet.reshape(1, DIM), batchf)


def kernel(x, edge_index, batch, W_pre, b_pre,
           W1_0, b1_0, W2_0, b2_0, gam_0, bet_0,
           W1_1, b1_1, W2_1, b2_1, gam_1, bet_1,
           W1_2, b1_2, W2_2, b2_2, gam_2, bet_2):
    src = edge_index[0]
    dst = edge_index[1]
    batchf = batch.reshape(1, N)
    zeros = jnp.zeros((LAST, DIM), jnp.float32)
    layers = [
        (W1_0, b1_0, W2_0, b2_0, gam_0, bet_0),
        (W1_1, b1_1, W2_1, b2_1, gam_1, bet_1),
        (W1_2, b1_2, W2_2, b2_2, gam_2, bet_2),
    ]
    h = _pre_linear(x, W_pre, b_pre)
    xs, pools = [], []
    for (W1, b1, W2, b2, gam, bet) in layers:
        agg = _segment_sum(h, src, dst, zeros)
        h, p = _layer(h, agg, W1, b1, W2, b2, gam, bet, batchf)
        xs.append(h)
        pools.append(p)
    return (jnp.concatenate(pools, axis=1), jnp.concatenate(xs, axis=1))
